# baseline (speedup 1.0000x reference)
import jax
import jax.numpy as jnp
from jax.experimental import pallas as pl

N_NODES = 10000
N_EDGES = 320000
HEADS = 4
HID = 128
GAT_OUT = 64
NUM_GRAPHS = 256
NEG_SLOPE = 0.2


def _tags(pm):
    b = jnp.concatenate([pm[1:] != pm[:-1], jnp.array([True])])
    return b.astype(jnp.float32)


def _extend(edge_index, edge_attr, n):
    src, dst = edge_index[0], edge_index[1]
    valid = src != dst
    vm = valid.astype(jnp.float32)
    loop_sum = jax.ops.segment_sum(edge_attr * vm[:, None], dst, num_segments=n)
    loop_cnt = jax.ops.segment_sum(vm, dst, num_segments=n)
    loop_attr = loop_sum / jnp.maximum(loop_cnt, 1.0)[:, None]
    ar = jnp.arange(n)
    src2 = jnp.concatenate([src, ar])
    dst2 = jnp.concatenate([dst, ar])
    valid2 = jnp.concatenate([valid, jnp.ones((n,), bool)])
    attr2 = jnp.concatenate([edge_attr, loop_attr], axis=0)
    return src2, dst2, valid2, attr2


def _gat(x, src, dst, valid, e_attr, W, a_s, a_d, W_e, a_e, b, H, C, n):
    xl = (x @ W).reshape(-1, H, C)
    el = (e_attr @ W_e).reshape(-1, H, C)
    alpha = (xl[src] * a_s).sum(-1) + (xl[dst] * a_d).sum(-1) + (el * a_e).sum(-1)
    alpha = jax.nn.leaky_relu(alpha, NEG_SLOPE)
    alpha = jnp.where(valid[:, None], alpha, -jnp.inf)
    amax = jax.ops.segment_max(alpha, dst, num_segments=n)
    amax = jnp.where(jnp.isfinite(amax), amax, 0.0)
    ex = jnp.where(valid[:, None], jnp.exp(alpha - amax[dst]), 0.0)
    den = jax.ops.segment_sum(ex, dst, num_segments=n)
    att = ex / (den[dst] + 1e-16)
    out = jax.ops.segment_sum(xl[src] * att[:, :, None], dst, num_segments=n)
    return out.reshape(n, H * C) + b


def _pool(x, pm, m, g):
    s = jax.ops.segment_sum(x * m[:, None], pm, num_segments=g)
    c = jax.ops.segment_sum(m, pm, num_segments=g)
    return s / jnp.maximum(c, 1.0)[:, None]


def _concat_kernel(a_ref, b_ref, o_ref):
    o_ref[:, :GAT_OUT] = a_ref[...]
    o_ref[:, GAT_OUT:] = b_ref[...]


def kernel(node_features, edge_index, edge_attr, polymer_mapping, W1, att_src1,
           att_dst1, We1, att_e1, b1, W2, att_src2, att_dst2, We2, att_e2, b2):
    n = node_features.shape[0]
    tags = _tags(polymer_mapping)
    x = jnp.concatenate([node_features, tags[:, None]], axis=-1)
    src, dst, valid, eattr = _extend(edge_index, edge_attr, n)
    x = _gat(x, src, dst, valid, eattr, W1, att_src1, att_dst1, We1, att_e1, b1,
             HEADS, HID, n)
    x = jax.nn.relu(x)
    x = _gat(x, src, dst, valid, eattr, W2, att_src2, att_dst2, We2, att_e2, b2,
             1, GAT_OUT, n)
    mono = _pool(x, polymer_mapping, 1.0 - tags, NUM_GRAPHS)
    solv = _pool(x, polymer_mapping, tags, NUM_GRAPHS)
    return pl.pallas_call(
        _concat_kernel,
        out_shape=jax.ShapeDtypeStruct((NUM_GRAPHS, 2 * GAT_OUT), jnp.float32),
    )(mono, solv)


# speedup vs baseline: 9.2396x; 9.2396x over previous
import functools

import jax
import jax.numpy as jnp
from jax import lax
from jax.experimental import pallas as pl
from jax.experimental.pallas import tpu as pltpu
from jax.experimental.pallas import tpu_sc as plsc

N = 10000
NP = 10240
TB = NP // 128
E = 320000
EP = 327680
H = 4
C = 128
C2 = 64
G = 256
SLOPE = 0.2

NC = 2
NS = 16
NW = NC * NS
EWP = EP // NW
NSLICE = NP // NS
OC = 2048
K = 128

_mesh = functools.partial(
    plsc.VectorSubcoreMesh, core_axis_name="c", subcore_axis_name="s",
    num_cores=NC, num_subcores=NS)
_sc_params = pltpu.CompilerParams(needs_layout_passes=False)


def _leaky(a):
    return jnp.where(a >= 0, a, a * SLOPE)


def _tca_body(nf_ref, pm_ref, w1_ref, as_ref, ad_ref,
              xltab_ref, a_src_ref, a_dst_ref, tags_ref):
    pm = pm_ref[...]
    nxt = jnp.concatenate([pm[1:], jnp.full((1,), -1, jnp.int32)])
    tags = (pm != nxt).astype(jnp.float32)
    tags_ref[...] = tags
    nf = nf_ref[...]
    for h in range(H):
        wh = w1_ref[:, pl.ds(h * C, C)]
        xlh = jnp.dot(nf, wh[:128, :], preferred_element_type=jnp.float32)
        xlh = xlh + tags[:, None] * wh[128, :][None, :]
        xltab_ref[pl.ds(h * NP, N), :] = xlh
        a_src_ref[:, pl.ds(h, 1)] = jnp.sum(
            xlh * as_ref[0, h, :][None, :], axis=1, keepdims=True)
        a_dst_ref[:, pl.ds(h, 1)] = jnp.sum(
            xlh * ad_ref[0, h, :][None, :], axis=1, keepdims=True)


def _tca(nf, pm, w1, att_src1, att_dst1):
    return pl.pallas_call(
        _tca_body,
        out_shape=[
            jax.ShapeDtypeStruct((H * NP, C), jnp.float32),
            jax.ShapeDtypeStruct((N, H), jnp.float32),
            jax.ShapeDtypeStruct((N, H), jnp.float32),
            jax.ShapeDtypeStruct((N,), jnp.float32),
        ],
    )(nf, pm, w1, att_src1, att_dst1)


def _tca2_body(src_ref, dst_ref, ea_ref, we1_ref, ae1_ref, we2_ref, ae2_ref,
               *outs):
    et1 = outs[:H]
    et2_ref, sv0_ref, sv1_ref, sv2_ref, srcp_ref, dstp_ref = outs[H:]
    src = src_ref[...]
    dst = dst_ref[...]
    vm = (src != dst).astype(jnp.float32)
    a0 = ea_ref[0, :]
    a1 = ea_ref[1, :]
    zpad = jnp.zeros((EP - E,), jnp.float32)
    for h in range(H):
        b0 = jnp.sum(we1_ref[0, pl.ds(h * C, C)] * ae1_ref[0, h, :])
        b1 = jnp.sum(we1_ref[1, pl.ds(h * C, C)] * ae1_ref[0, h, :])
        et1[h][pl.ds(0, E)] = a0 * b0 + a1 * b1
        et1[h][pl.ds(E, EP - E)] = zpad
    c0 = jnp.sum(we2_ref[0, :] * ae2_ref[0, 0, :])
    c1 = jnp.sum(we2_ref[1, :] * ae2_ref[0, 0, :])
    et2_ref[pl.ds(0, E)] = a0 * c0 + a1 * c1
    et2_ref[pl.ds(E, EP - E)] = zpad
    sv0_ref[pl.ds(0, E)] = a0 * vm
    sv0_ref[pl.ds(E, EP - E)] = zpad
    sv1_ref[pl.ds(0, E)] = a1 * vm
    sv1_ref[pl.ds(E, EP - E)] = zpad
    sv2_ref[pl.ds(0, E)] = vm
    sv2_ref[pl.ds(E, EP - E)] = zpad
    zpadi = jnp.zeros((EP - E,), jnp.int32)
    srcp_ref[pl.ds(0, E)] = src
    srcp_ref[pl.ds(E, EP - E)] = zpadi
    dstp_ref[pl.ds(0, E)] = dst
    dstp_ref[pl.ds(E, EP - E)] = zpadi


def _tca2(srcv, dstv, ea, we1, ae1, we2, ae2):
    fvec = jax.ShapeDtypeStruct((EP,), jnp.float32)
    ivec = jax.ShapeDtypeStruct((EP,), jnp.int32)
    return pl.pallas_call(
        _tca2_body,
        out_shape=[fvec] * H + [fvec, fvec, fvec, fvec, ivec, ivec],
    )(srcv, dstv, ea, we1, ae1, we2, ae2)


def _tcb_body(lacc_ref, asd_ref, we1_ref, ae1_ref, we2_ref, ae2_ref,
              exl_ref, etl_ref):
    accsum = jnp.sum(lacc_ref[...], axis=0)
    s0 = accsum[0:TB, :]
    s1 = accsum[TB:2 * TB, :]
    s2 = accsum[2 * TB:3 * TB, :]
    cnt = jnp.maximum(s2, 1.0)
    la0 = s0 / cnt
    la1 = s1 / cnt
    for h in range(H):
        b0 = jnp.sum(we1_ref[0, pl.ds(h * C, C)] * ae1_ref[0, h, :])
        b1 = jnp.sum(we1_ref[1, pl.ds(h * C, C)] * ae1_ref[0, h, :])
        asb = asd_ref[pl.ds(h * TB, TB), :]
        adb = asd_ref[pl.ds((H + h) * TB, TB), :]
        al = asb + adb + la0 * b0 + la1 * b1
        exl_ref[h, :, :] = jnp.exp(_leaky(al))
    c0 = jnp.sum(we2_ref[0, :] * ae2_ref[0, 0, :])
    c1 = jnp.sum(we2_ref[1, :] * ae2_ref[0, 0, :])
    etl_ref[...] = la0 * c0 + la1 * c1


def _tcb(lacc, asd1, we1, ae1, we2, ae2):
    return pl.pallas_call(
        _tcb_body,
        out_shape=[
            jax.ShapeDtypeStruct((H, TB, 128), jnp.float32),
            jax.ShapeDtypeStruct((TB, 128), jnp.float32),
        ],
    )(lacc, asd1, we1, ae1, we2, ae2)


def _tcds_body(denp_ref, out_ref):
    out_ref[...] = jnp.sum(denp_ref[...], axis=0)


def _tcds(denp):
    return pl.pallas_call(
        _tcds_body,
        out_shape=jax.ShapeDtypeStruct((H * TB, 128), jnp.float32),
    )(denp)


def _tcd_body(agg_ref, xltab_ref, exl_ref, den_ref, b1_ref, w2_ref, xl2_ref):
    h = pl.program_id(0)
    num = (agg_ref[0, 0, pl.ds(0, N), :] + agg_ref[1, 0, pl.ds(0, N), :])
    exh = exl_ref[0, 0, :][:, None]
    den = den_ref[0, 0, :][:, None] + exh
    num = num + exh * xltab_ref[pl.ds(0, N), :]
    out1 = num / (den + 1e-16) + b1_ref[...][None, :]
    out1 = jnp.maximum(out1, 0.0)
    contrib = jnp.dot(out1, w2_ref[...], preferred_element_type=jnp.float32)

    @pl.when(h == 0)
    def _():
        xl2_ref[...] = contrib

    @pl.when(h != 0)
    def _():
        xl2_ref[...] = xl2_ref[...] + contrib


def _tcd(agg1, xltab, exloop1, den1, b1, w2):
    return pl.pallas_call(
        _tcd_body,
        grid=(H,),
        in_specs=[
            pl.BlockSpec((NC, 1, NP, C), lambda h: (0, h, 0, 0)),
            pl.BlockSpec((NP, C), lambda h: (h, 0)),
            pl.BlockSpec((1, 1, N), lambda h: (h, 0, 0)),
            pl.BlockSpec((1, 1, N), lambda h: (h, 0, 0)),
            pl.BlockSpec((C,), lambda h: (h,)),
            pl.BlockSpec((C, C2), lambda h: (h, 0)),
        ],
        out_specs=pl.BlockSpec((N, C2), lambda h: (0, 0)),
        out_shape=jax.ShapeDtypeStruct((N, C2), jnp.float32),
    )(agg1, xltab, exloop1, den1, b1, w2)


def _tcd2_body(xl2_ref, as2_ref, ad2_ref, etl2_ref,
               tab_ref, a2s_ref, a2d_ref, exl2_ref):
    xl2 = xl2_ref[...]
    onescol = jnp.where(
        lax.broadcasted_iota(jnp.int32, (N, C2), 1) == 0, 1.0, 0.0)
    tab_ref[pl.ds(0, N), pl.ds(0, C2)] = xl2
    tab_ref[pl.ds(0, N), pl.ds(C2, C2)] = onescol
    a_s = jnp.sum(xl2 * as2_ref[0, 0, :][None, :], axis=1)
    a_d = jnp.sum(xl2 * ad2_ref[0, 0, :][None, :], axis=1)
    a2s_ref[...] = a_s
    a2d_ref[...] = a_d
    exl2_ref[...] = jnp.exp(_leaky(a_s + a_d + etl2_ref[...]))


def _tcd2(xl2, att_src2, att_dst2, etloop2):
    return pl.pallas_call(
        _tcd2_body,
        out_shape=[
            jax.ShapeDtypeStruct((NP, 128), jnp.float32),
            jax.ShapeDtypeStruct((N,), jnp.float32),
            jax.ShapeDtypeStruct((N,), jnp.float32),
            jax.ShapeDtypeStruct((N,), jnp.float32),
        ],
    )(xl2, att_src2, att_dst2, etloop2)


def _tce_body(agg_ref, exl2_ref, tab_ref, b2_ref, pm_ref, tags_ref, out_ref):
    exl2 = exl2_ref[...]
    tot = (agg_ref[0, pl.ds(0, N), :] + agg_ref[1, pl.ds(0, N), :]
           + exl2[:, None] * tab_ref[pl.ds(0, N), :])
    mask64 = (lax.broadcasted_iota(jnp.int32, (N, 128), 1) == C2)
    den = jnp.sum(jnp.where(mask64, tot, 0.0), axis=1, keepdims=True)
    out2 = tot[:, 0:C2] / (den + 1e-16) + b2_ref[...][None, :]
    tags = tags_ref[...]
    pm = pm_ref[...]
    onehot = (pm[None, :] == lax.broadcasted_iota(jnp.int32, (G, N), 0))
    onehot = onehot.astype(jnp.float32)
    mmono = onehot * (1.0 - tags)[None, :]
    msolv = onehot * tags[None, :]
    mono = jnp.dot(mmono, out2, preferred_element_type=jnp.float32)
    solv = jnp.dot(msolv, out2, preferred_element_type=jnp.float32)
    cmono = jnp.maximum(jnp.sum(mmono, axis=1), 1.0)
    csolv = jnp.maximum(jnp.sum(msolv, axis=1), 1.0)
    out_ref[:, pl.ds(0, C2)] = mono / cmono[:, None]
    out_ref[:, pl.ds(C2, C2)] = solv / csolv[:, None]


def _tce(agg2, exloop2, tab2, b2, pm, tags):
    return pl.pallas_call(
        _tce_body,
        out_shape=jax.ShapeDtypeStruct((G, 2 * C2), jnp.float32),
    )(agg2, exloop2, tab2, b2, pm, tags)


def _zero_fill(buf, nrow):
    def body(i, _):
        r = i // 8
        buf[r, pl.ds((i - r * 8) * 16, 16)] = jnp.zeros((16,), jnp.float32)
        return 0

    lax.fori_loop(0, nrow * 8, body, 0)


def _rc(idx16):
    return lax.shift_right_logical(idx16, 7), lax.bitwise_and(idx16, 127)


def _sc0_body(dst_ref, sv0_ref, sv1_ref, sv2_ref, out_ref,
              buf_v, didx_v, tab_v, sem):
    c = lax.axis_index("c")
    s = lax.axis_index("s")
    wid = c * NS + s
    _zero_fill(tab_v, 3 * TB)
    svs = [sv0_ref, sv1_ref, sv2_ref]

    def chunk(i, _):
        base = wid * EWP + i * 512
        for j in range(3):
            pltpu.sync_copy(svs[j].at[pl.ds(base, 512)],
                            buf_v.at[pl.ds(j * 512, 512)])
        pltpu.sync_copy(dst_ref.at[pl.ds(base, 512)], didx_v)

        def inner(k, _):
            off = k * 16
            d16 = didx_v[pl.ds(off, 16)]
            drow, dcol = _rc(d16)
            for j in range(3):
                val = buf_v[pl.ds(j * 512 + off, 16)]
                plsc.addupdate_scatter(tab_v, [drow + j * TB, dcol], val)
            return 0

        lax.fori_loop(0, 512 // 16, inner, 0)
        return 0

    lax.fori_loop(0, EWP // 512, chunk, 0)
    pltpu.sync_copy(tab_v, out_ref.at[wid])


def _sc0(dstp, sv0, sv1, sv2):
    kfn = pl.kernel(
        _sc0_body,
        out_type=jax.ShapeDtypeStruct((NW, 3 * TB, 128), jnp.float32),
        mesh=_mesh(),
        compiler_params=_sc_params,
        scratch_types=[
            pltpu.VMEM((3 * 512,), jnp.float32),
            pltpu.VMEM((512,), jnp.int32),
            pltpu.VMEM((3 * TB, 128), jnp.float32),
            pltpu.SemaphoreType.DMA,
        ],
    )
    return kfn(dstp, sv0, sv1, sv2)


def _make_scb_body(nh):
    def body(*refs):
        src_ref, dst_ref, asd_ref = refs[:3]
        ets = refs[3:3 + nh]
        exs = refs[3 + nh:3 + 2 * nh]
        tabs_v, src_v, dst_v, et_v, ex_v, sem = refs[3 + 2 * nh:]
        c = lax.axis_index("c")
        s = lax.axis_index("s")
        wid = c * NS + s
        pltpu.sync_copy(asd_ref, tabs_v)

        def outer(i, _):
            base = wid * EWP + i * OC
            pltpu.sync_copy(src_ref.at[pl.ds(base, OC)], src_v)
            pltpu.sync_copy(dst_ref.at[pl.ds(base, OC)], dst_v)
            for h in range(nh):
                pltpu.sync_copy(ets[h].at[pl.ds(base, OC)],
                                et_v.at[pl.ds(h * OC, OC)])

            def inner(k, _):
                off = k * 16
                s16 = src_v[pl.ds(off, 16)]
                d16 = dst_v[pl.ds(off, 16)]
                valid = s16 != d16
                srow, scol = _rc(s16)
                drow, dcol = _rc(d16)
                for h in range(nh):
                    asg = plsc.load_gather(tabs_v, [srow + h * TB, scol])
                    adg = plsc.load_gather(tabs_v, [drow + (nh + h) * TB, dcol])
                    a = asg + adg + et_v[pl.ds(h * OC + off, 16)]
                    exv = jnp.where(valid, jnp.exp(_leaky(a)), 0.0)
                    ex_v[pl.ds(h * OC + off, 16)] = exv
                return 0

            lax.fori_loop(0, OC // 16, inner, 0)
            for h in range(nh):
                pltpu.sync_copy(ex_v.at[pl.ds(h * OC, OC)],
                                exs[h].at[pl.ds(base, OC)])
            return 0

        lax.fori_loop(0, EWP // OC, outer, 0)

    return body


def _scb(srcp, dstp, asd, ets, nh):
    kfn = pl.kernel(
        _make_scb_body(nh),
        out_type=[jax.ShapeDtypeStruct((EP,), jnp.float32)] * nh,
        mesh=_mesh(),
        compiler_params=_sc_params,
        scratch_types=[
            pltpu.VMEM((2 * nh * TB, 128), jnp.float32),
            pltpu.VMEM((OC,), jnp.int32),
            pltpu.VMEM((OC,), jnp.int32),
            pltpu.VMEM((nh * OC,), jnp.float32),
            pltpu.VMEM((nh * OC,), jnp.float32),
            pltpu.SemaphoreType.DMA,
        ],
    )
    return kfn(srcp, dstp, asd, *ets)


def _make_scc_body(nh, with_den):
    def body(*refs):
        src_ref, dst_ref, tab_ref = refs[:3]
        exs = refs[3:3 + nh]
        if with_den:
            out_ref, den_ref = refs[3 + nh:5 + nh]
            scr = refs[5 + nh:]
        else:
            out_ref = refs[3 + nh]
            scr = refs[4 + nh:]
        rows_v, sidx_v, gidx_v, didx_v, ex_v, den_v, zbuf, sem, acc_sh = scr
        c = lax.axis_index("c")
        s = lax.axis_index("s")
        wid = c * NS + s
        iota16 = lax.iota(jnp.int32, 16)
        _zero_fill(zbuf, K)
        for h in range(nh):
            for t in range(NSLICE // K):
                pltpu.sync_copy(zbuf, acc_sh.at[pl.ds(s * NSLICE + t * K, K)])
            if with_den:
                _zero_fill(den_v, TB)
            plsc.subcore_barrier()

            def chunk(i, _):
                base = wid * EWP + i * K
                pltpu.sync_copy(src_ref.at[pl.ds(base, K)], sidx_v)
                pltpu.sync_copy(dst_ref.at[pl.ds(base, K)], didx_v)
                pltpu.sync_copy(exs[h].at[pl.ds(base, K)], ex_v)
                if nh > 1:
                    for j in range(K // 16):
                        gidx_v[pl.ds(j * 16, 16)] = (
                            sidx_v[pl.ds(j * 16, 16)] + h * NP)
                    idx_ref = gidx_v
                else:
                    idx_ref = sidx_v
                pltpu.async_copy(tab_ref.at[idx_ref], rows_v, sem).wait()

                def scale(r, _):
                    m = plsc.load_gather(ex_v, [jnp.full((16,), r, jnp.int32)])
                    ri = jnp.full((16,), r, jnp.int32)
                    for j in range(8):
                        ci = iota16 + j * 16
                        v = plsc.load_gather(rows_v, [ri, ci])
                        plsc.store_scatter(rows_v, [ri, ci], v * m)
                    return 0

                lax.fori_loop(0, K, scale, 0)
                pltpu.sync_copy(rows_v, acc_sh.at[didx_v], add=True)
                if with_den:
                    for j in range(K // 16):
                        d16 = didx_v[pl.ds(j * 16, 16)]
                        e16 = ex_v[pl.ds(j * 16, 16)]
                        drow, dcol = _rc(d16)
                        plsc.addupdate_scatter(den_v, [drow, dcol], e16)
                return 0

            lax.fori_loop(0, EWP // K, chunk, 0)
            plsc.subcore_barrier()
            if nh > 1:
                pltpu.sync_copy(acc_sh.at[pl.ds(s * NSLICE, NSLICE)],
                                out_ref.at[c, h, pl.ds(s * NSLICE, NSLICE)])
            else:
                pltpu.sync_copy(acc_sh.at[pl.ds(s * NSLICE, NSLICE)],
                                out_ref.at[c, pl.ds(s * NSLICE, NSLICE)])
            if with_den:
                pltpu.sync_copy(den_v, den_ref.at[wid, pl.ds(h * TB, TB)])

    return body


def _scc(srcp, dstp, tab, exs, nh, with_den):
    out_types = [jax.ShapeDtypeStruct(
        (NC, nh, NP, 128) if nh > 1 else (NC, NP, 128), jnp.float32)]
    if with_den:
        out_types.append(
            jax.ShapeDtypeStruct((NW, nh * TB, 128), jnp.float32))
    kfn = pl.kernel(
        _make_scc_body(nh, with_den),
        out_type=out_types,
        mesh=_mesh(),
        compiler_params=_sc_params,
        scratch_types=[
            pltpu.VMEM((K, 128), jnp.float32),
            pltpu.VMEM((K,), jnp.int32),
            pltpu.VMEM((K,), jnp.int32),
            pltpu.VMEM((K,), jnp.int32),
            pltpu.VMEM((K,), jnp.float32),
            pltpu.VMEM((TB, 128), jnp.float32),
            pltpu.VMEM((K, 128), jnp.float32),
            pltpu.SemaphoreType.DMA,
            pltpu.VMEM_SHARED((NP, 128), jnp.float32),
        ],
    )
    return kfn(srcp, dstp, tab, *exs)


def _to_tables(cols):
    mat = jnp.stack([jnp.pad(v, (0, NP - N)) for v in cols], axis=0)
    return mat.reshape(len(cols) * TB, 128)


def kernel(node_features, edge_index, edge_attr, polymer_mapping,
           W1, att_src1, att_dst1, We1, att_e1, b1,
           W2, att_src2, att_dst2, We2, att_e2, b2):
    ei = edge_index.astype(jnp.int32)
    pm = polymer_mapping.astype(jnp.int32)
    srcv = ei[0]
    dstv = ei[1]

    xltab, as1, ad1, tags = _tca(node_features, pm, W1, att_src1, att_dst1)
    (et1a, et1b, et1c, et1d, et2, sv0, sv1, sv2, srcp, dstp) = _tca2(
        srcv, dstv, edge_attr.T, We1, att_e1, We2, att_e2)
    lacc = _sc0(dstp, sv0, sv1, sv2)

    asd1 = _to_tables([as1[:, h] for h in range(H)]
                      + [ad1[:, h] for h in range(H)])
    exl1b, etl2b = _tcb(lacc, asd1, We1, att_e1, We2, att_e2)
    exloop1 = exl1b.reshape(H, 1, NP)[:, :, :N]
    etloop2 = etl2b.reshape(NP)[:N]

    ex1 = _scb(srcp, dstp, asd1, [et1a, et1b, et1c, et1d], H)
    agg1, den1p = _scc(srcp, dstp, xltab, ex1, H, True)
    den1 = _tcds(den1p).reshape(H, 1, NP)[:, :, :N]
    xl2 = _tcd(agg1, xltab, exloop1, den1, b1, W2)
    tab2, as2, ad2, exloop2 = _tcd2(xl2, att_src2, att_dst2, etloop2)

    asd2 = _to_tables([as2, ad2])
    (ex2,) = _scb(srcp, dstp, asd2, [et2], 1)
    (agg2,) = _scc(srcp, dstp, tab2, [ex2], 1, False)
    return _tce(agg2, exloop2, tab2, b2, pm, tags)
